# baseline (speedup 1.0000x reference)
import functools

import jax
import jax.numpy as jnp
from jax import lax
from jax.experimental import pallas as pl
from jax.experimental.pallas import tpu as pltpu
from jax.experimental.pallas import tpu_sc as plsc

N = 10000
E = 320000
D = 128

NC = 2
NS = 16
NW = NC * NS
CH = 88
CPW0 = 102
CPW1 = 126
CPWMX = max(CPW0, CPW1)
NCHUNK = NS * (CPW0 + CPW1)
E_PAD = NCHUNK * CH
M_ROWS = 10016
BS = M_ROWS // 2
TILE_ROWS = 632
LAST_ROWS = N - (NS - 1) * TILE_ROWS
NB = 3
PF = 2


def _seg_sum_sc(m, src3d, dst3d):
    mesh = plsc.VectorSubcoreMesh(core_axis_name="c", subcore_axis_name="s")

    @functools.partial(
        pl.kernel,
        out_type=(
            jax.ShapeDtypeStruct((N, D), jnp.float32),
            jax.ShapeDtypeStruct((N, D), jnp.float32),
        ),
        mesh=mesh,
        scratch_types=[
            pltpu.VMEM((CPWMX, CH), jnp.int32),
            pltpu.VMEM((NB, CH), jnp.int32),
            pltpu.VMEM((NB, CH, D), jnp.float32),
            pltpu.VMEM_SHARED((N, D), jnp.float32),
        ] + [pltpu.SemaphoreType.DMA] * (2 * NB),
    )
    def seg_kernel(m_hbm, src_hbm, dst_hbm, p0_hbm, p1_hbm,
                   src_v, dst_v, rows_v, acc_sh, *sems):
        semg = sems[:NB]
        semd = sems[NB:2 * NB]
        c = lax.axis_index("c")
        s = lax.axis_index("s")
        wid = c * NS + s
        cbase = wid * CPWMX
        base = s * TILE_ROWS

        @pl.loop(0, CH)
        def _zero_rows(i):
            @pl.loop(0, D // 16)
            def _zero_lanes(k16):
                rows_v[0, i, pl.ds(k16 * 16, 16)] = jnp.zeros((16,),
                                                              jnp.float32)

        @pl.when(s < NS - 1)
        def _zero_full_tile():
            @pl.loop(0, TILE_ROWS // CH)
            def _zero_acc(t):
                pltpu.sync_copy(rows_v.at[0],
                                acc_sh.at[pl.ds(base + t * CH, CH)])
            pltpu.sync_copy(
                rows_v.at[0].at[pl.ds(0, TILE_ROWS % CH)],
                acc_sh.at[pl.ds(base + (TILE_ROWS // CH) * CH,
                                TILE_ROWS % CH)])

        @pl.when(s == NS - 1)
        def _zero_last_tile():
            @pl.loop(0, LAST_ROWS // CH)
            def _zero_acc(t):
                pltpu.sync_copy(rows_v.at[0],
                                acc_sh.at[pl.ds(base + t * CH, CH)])
            pltpu.sync_copy(
                rows_v.at[0].at[pl.ds(0, LAST_ROWS % CH)],
                acc_sh.at[pl.ds(base + (LAST_ROWS // CH) * CH,
                                LAST_ROWS % CH)])

        pltpu.sync_copy(src_hbm.at[wid], src_v)
        for j in range(PF):
            pltpu.async_copy(dst_hbm.at[cbase + j].at[0], dst_v.at[j],
                             semd[j])
            pltpu.async_copy(m_hbm.at[src_v.at[j]], rows_v.at[j], semg[j])
        plsc.subcore_barrier()

        def edge_loop(cpw):
            @pl.loop(0, cpw, step=NB)
            def _edges(t):
                for k in range(NB):
                    b = k
                    bn = (k + PF) % NB
                    cur = t + k
                    pltpu.make_async_copy(
                        m_hbm.at[src_v.at[cur]], rows_v.at[b],
                        semg[b]).wait()

                    @pl.when(cur + PF < cpw)
                    def _issue_gather():
                        pltpu.async_copy(
                            m_hbm.at[src_v.at[cur + PF]], rows_v.at[bn],
                            semg[bn])

                    pltpu.make_async_copy(
                        dst_hbm.at[cbase + cur].at[0], dst_v.at[b],
                        semd[b]).wait()
                    pltpu.sync_copy(rows_v.at[b], acc_sh.at[dst_v.at[b]],
                                    add=True)

                    @pl.when(cur + PF < cpw)
                    def _issue_dst():
                        pltpu.async_copy(
                            dst_hbm.at[cbase + cur + PF].at[0],
                            dst_v.at[bn], semd[bn])

        @pl.when(c == 0)
        def _edges_c0():
            edge_loop(CPW0)

        @pl.when(c == 1)
        def _edges_c1():
            edge_loop(CPW1)
        plsc.subcore_barrier()

        @pl.when(jnp.logical_and(c == 0, s < NS - 1))
        def _out0():
            pltpu.sync_copy(acc_sh.at[pl.ds(base, TILE_ROWS)],
                            p0_hbm.at[pl.ds(base, TILE_ROWS)])

        @pl.when(jnp.logical_and(c == 0, s == NS - 1))
        def _out0_last():
            pltpu.sync_copy(acc_sh.at[pl.ds(base, LAST_ROWS)],
                            p0_hbm.at[pl.ds(base, LAST_ROWS)])

        @pl.when(jnp.logical_and(c == 1, s < NS - 1))
        def _out1():
            pltpu.sync_copy(acc_sh.at[pl.ds(base, TILE_ROWS)],
                            p1_hbm.at[pl.ds(base, TILE_ROWS)])

        @pl.when(jnp.logical_and(c == 1, s == NS - 1))
        def _out1_last():
            pltpu.sync_copy(acc_sh.at[pl.ds(base, LAST_ROWS)],
                            p1_hbm.at[pl.ds(base, LAST_ROWS)])

    return seg_kernel(m, src3d, dst3d)


def _row_mask(shape):
    i = pl.program_id(0)
    rows = i * BS + lax.broadcasted_iota(jnp.int32, shape, 0)
    return rows < N


def _mm_body(h, wr_ref, wo_ref, b_ref, m_ref, r_ref):
    dn = (((1,), (1,)), ((), ()))
    m_ref[...] = lax.dot_general(
        h, wr_ref[...], dn, preferred_element_type=jnp.float32,
        precision=lax.Precision.HIGHEST)
    r_ref[...] = lax.dot_general(
        h, wo_ref[...], dn, preferred_element_type=jnp.float32,
        precision=lax.Precision.HIGHEST) + b_ref[...]


def _tc_pre(x, W_rel, W_root, b):
    def body(x_ref, wr_ref, wo_ref, b_ref, m_ref, r_ref):
        h = jnp.where(_row_mask((BS, D)), x_ref[...], 0.0)
        _mm_body(h, wr_ref, wo_ref, b_ref, m_ref, r_ref)

    return pl.pallas_call(
        body,
        grid=(2,),
        in_specs=[
            pl.BlockSpec((BS, D), lambda i: (i, 0)),
            pl.BlockSpec((D, D), lambda i: (0, 0)),
            pl.BlockSpec((D, D), lambda i: (0, 0)),
            pl.BlockSpec((1, D), lambda i: (0, 0)),
        ],
        out_specs=[
            pl.BlockSpec((BS, D), lambda i: (i, 0)),
            pl.BlockSpec((BS, D), lambda i: (i, 0)),
        ],
        out_shape=(jax.ShapeDtypeStruct((M_ROWS, D), jnp.float32),
                   jax.ShapeDtypeStruct((N, D), jnp.float32)),
    )(x, W_rel, W_root, b.reshape(1, D))


def _tc_mid(p0, p1, r, W_rel, W_root, b):
    def body(p0_ref, p1_ref, r_ref, wr_ref, wo_ref, b_ref, m_ref, r2_ref):
        h = jnp.maximum(p0_ref[...] + p1_ref[...] + r_ref[...], 0.0)
        h = jnp.where(_row_mask((BS, D)), h, 0.0)
        _mm_body(h, wr_ref, wo_ref, b_ref, m_ref, r2_ref)

    return pl.pallas_call(
        body,
        grid=(2,),
        in_specs=[
            pl.BlockSpec((BS, D), lambda i: (i, 0)),
            pl.BlockSpec((BS, D), lambda i: (i, 0)),
            pl.BlockSpec((BS, D), lambda i: (i, 0)),
            pl.BlockSpec((D, D), lambda i: (0, 0)),
            pl.BlockSpec((D, D), lambda i: (0, 0)),
            pl.BlockSpec((1, D), lambda i: (0, 0)),
        ],
        out_specs=[
            pl.BlockSpec((BS, D), lambda i: (i, 0)),
            pl.BlockSpec((BS, D), lambda i: (i, 0)),
        ],
        out_shape=(jax.ShapeDtypeStruct((M_ROWS, D), jnp.float32),
                   jax.ShapeDtypeStruct((N, D), jnp.float32)),
    )(p0, p1, r, W_rel, W_root, b.reshape(1, D))


def _tc_final(p0, p1, r):
    def body(p0_ref, p1_ref, r_ref, o_ref):
        o_ref[...] = p0_ref[...] + p1_ref[...] + r_ref[...]

    return pl.pallas_call(
        body,
        out_shape=jax.ShapeDtypeStruct((N, D), jnp.float32),
    )(p0, p1, r)


def _make_chunk_map():
    import numpy as np
    cmap = np.zeros((NW, CPWMX), dtype=np.int32)
    pos = 0
    for c in range(NC):
        cpw = CPW0 if c == 0 else CPW1
        for s in range(NS):
            w = c * NS + s
            cmap[w, :cpw] = np.arange(pos, pos + cpw, dtype=np.int32)
            pos += cpw
    assert pos == NCHUNK
    return cmap


_CHUNK_MAP = jnp.asarray(_make_chunk_map())


def kernel(x, relationsedge_indices_relations, W_rel1, b_rel1, W_root1,
           W_rel2, b_rel2, W_root2):
    ei = relationsedge_indices_relations[-1]
    src, dst = ei[0], ei[1]
    pad = E_PAD - E
    src_flat = jnp.concatenate(
        [src, jnp.full((pad,), N, jnp.int32)]).reshape(NCHUNK, CH)
    dst_flat = jnp.concatenate(
        [dst, jnp.zeros((pad,), jnp.int32)]).reshape(NCHUNK, CH)
    src3d = jnp.take(src_flat, _CHUNK_MAP.reshape(-1), axis=0).reshape(
        NW, CPWMX, CH)
    dst3d = jnp.take(dst_flat, _CHUNK_MAP.reshape(-1), axis=0).reshape(
        NW * CPWMX, 1, CH)

    m1, r1 = _tc_pre(x, W_rel1, W_root1, b_rel1)
    p0, p1 = _seg_sum_sc(m1, src3d, dst3d)
    m2, r2 = _tc_mid(p0, p1, r1, W_rel2, W_root2, b_rel2)
    q0, q1 = _seg_sum_sc(m2, src3d, dst3d)
    out = _tc_final(q0, q1, r2)
    return out.reshape(N, 1, D)


# speedup vs baseline: 1.1703x; 1.1703x over previous
import functools

import jax
import jax.numpy as jnp
from jax import lax
from jax.experimental import pallas as pl
from jax.experimental.pallas import tpu as pltpu
from jax.experimental.pallas import tpu_sc as plsc

N = 10000
E = 320000
D = 128

NC = 2
NS = 16
NW = NC * NS
CH = 88
CPW = 114
NCHUNK = NW * CPW
E_PAD = NCHUNK * CH
M_ROWS = 10016
BS = M_ROWS // 2
TILE_ROWS = 632
LAST_ROWS = N - (NS - 1) * TILE_ROWS
NB = 3
PF = 2


def _seg_sum_sc(m, src3d, dst3d):
    mesh = plsc.VectorSubcoreMesh(core_axis_name="c", subcore_axis_name="s")

    @functools.partial(
        pl.kernel,
        out_type=(
            jax.ShapeDtypeStruct((N, D), jnp.float32),
            jax.ShapeDtypeStruct((N, D), jnp.float32),
        ),
        mesh=mesh,
        scratch_types=[
            pltpu.VMEM((CPW, CH), jnp.int32),
            pltpu.VMEM((NB, CH), jnp.int32),
            pltpu.VMEM((NB, CH, D), jnp.float32),
            pltpu.VMEM_SHARED((N, D), jnp.float32),
        ] + [pltpu.SemaphoreType.DMA] * (2 * NB),
    )
    def seg_kernel(m_hbm, src_hbm, dst_hbm, p0_hbm, p1_hbm,
                   src_v, dst_v, rows_v, acc_sh, *sems):
        semg = sems[:NB]
        semd = sems[NB:2 * NB]
        c = lax.axis_index("c")
        s = lax.axis_index("s")
        wid = c * NS + s
        cbase = wid * CPW
        base = s * TILE_ROWS

        @pl.loop(0, CH)
        def _zero_rows(i):
            @pl.loop(0, D // 16)
            def _zero_lanes(k16):
                rows_v[0, i, pl.ds(k16 * 16, 16)] = jnp.zeros((16,),
                                                              jnp.float32)

        @pl.when(s < NS - 1)
        def _zero_full_tile():
            @pl.loop(0, TILE_ROWS // CH)
            def _zero_acc(t):
                pltpu.sync_copy(rows_v.at[0],
                                acc_sh.at[pl.ds(base + t * CH, CH)])
            pltpu.sync_copy(
                rows_v.at[0].at[pl.ds(0, TILE_ROWS % CH)],
                acc_sh.at[pl.ds(base + (TILE_ROWS // CH) * CH,
                                TILE_ROWS % CH)])

        @pl.when(s == NS - 1)
        def _zero_last_tile():
            @pl.loop(0, LAST_ROWS // CH)
            def _zero_acc(t):
                pltpu.sync_copy(rows_v.at[0],
                                acc_sh.at[pl.ds(base + t * CH, CH)])
            pltpu.sync_copy(
                rows_v.at[0].at[pl.ds(0, LAST_ROWS % CH)],
                acc_sh.at[pl.ds(base + (LAST_ROWS // CH) * CH,
                                LAST_ROWS % CH)])

        pltpu.sync_copy(src_hbm.at[wid], src_v)
        for j in range(PF):
            pltpu.async_copy(dst_hbm.at[cbase + j].at[0], dst_v.at[j],
                             semd[j])
            pltpu.async_copy(m_hbm.at[src_v.at[j]], rows_v.at[j], semg[j])
        plsc.subcore_barrier()

        @pl.loop(0, CPW, step=NB)
        def _edges(t):
            for k in range(NB):
                b = k
                bn = (k + PF) % NB
                cur = t + k
                pltpu.make_async_copy(
                    m_hbm.at[src_v.at[cur]], rows_v.at[b], semg[b]).wait()

                @pl.when(cur + PF < CPW)
                def _issue_gather():
                    pltpu.async_copy(
                        m_hbm.at[src_v.at[cur + PF]], rows_v.at[bn],
                        semg[bn])

                pltpu.make_async_copy(
                    dst_hbm.at[cbase + cur].at[0], dst_v.at[b],
                    semd[b]).wait()
                pltpu.sync_copy(rows_v.at[b], acc_sh.at[dst_v.at[b]],
                                add=True)

                @pl.when(cur + PF < CPW)
                def _issue_dst():
                    pltpu.async_copy(
                        dst_hbm.at[cbase + cur + PF].at[0],
                        dst_v.at[bn], semd[bn])
        plsc.subcore_barrier()

        @pl.when(jnp.logical_and(c == 0, s < NS - 1))
        def _out0():
            pltpu.sync_copy(acc_sh.at[pl.ds(base, TILE_ROWS)],
                            p0_hbm.at[pl.ds(base, TILE_ROWS)])

        @pl.when(jnp.logical_and(c == 0, s == NS - 1))
        def _out0_last():
            pltpu.sync_copy(acc_sh.at[pl.ds(base, LAST_ROWS)],
                            p0_hbm.at[pl.ds(base, LAST_ROWS)])

        @pl.when(jnp.logical_and(c == 1, s < NS - 1))
        def _out1():
            pltpu.sync_copy(acc_sh.at[pl.ds(base, TILE_ROWS)],
                            p1_hbm.at[pl.ds(base, TILE_ROWS)])

        @pl.when(jnp.logical_and(c == 1, s == NS - 1))
        def _out1_last():
            pltpu.sync_copy(acc_sh.at[pl.ds(base, LAST_ROWS)],
                            p1_hbm.at[pl.ds(base, LAST_ROWS)])

    return seg_kernel(m, src3d, dst3d)


def _row_mask(shape):
    i = pl.program_id(0)
    rows = i * BS + lax.broadcasted_iota(jnp.int32, shape, 0)
    return rows < N


def _mm_body(h, wr_ref, wo_ref, b_ref, m_ref, r_ref):
    dn = (((1,), (1,)), ((), ()))
    m_ref[...] = lax.dot_general(
        h, wr_ref[...], dn, preferred_element_type=jnp.float32,
        precision=lax.Precision.HIGHEST)
    r_ref[...] = lax.dot_general(
        h, wo_ref[...], dn, preferred_element_type=jnp.float32,
        precision=lax.Precision.HIGHEST) + b_ref[...]


def _tc_pre(x, W_rel, W_root, b):
    def body(x_ref, wr_ref, wo_ref, b_ref, m_ref, r_ref):
        h = jnp.where(_row_mask((BS, D)), x_ref[...], 0.0)
        _mm_body(h, wr_ref, wo_ref, b_ref, m_ref, r_ref)

    return pl.pallas_call(
        body,
        grid=(2,),
        in_specs=[
            pl.BlockSpec((BS, D), lambda i: (i, 0)),
            pl.BlockSpec((D, D), lambda i: (0, 0)),
            pl.BlockSpec((D, D), lambda i: (0, 0)),
            pl.BlockSpec((1, D), lambda i: (0, 0)),
        ],
        out_specs=[
            pl.BlockSpec((BS, D), lambda i: (i, 0)),
            pl.BlockSpec((BS, D), lambda i: (i, 0)),
        ],
        out_shape=(jax.ShapeDtypeStruct((M_ROWS, D), jnp.float32),
                   jax.ShapeDtypeStruct((N, D), jnp.float32)),
    )(x, W_rel, W_root, b.reshape(1, D))


def _tc_mid(p0, p1, r, W_rel, W_root, b):
    def body(p0_ref, p1_ref, r_ref, wr_ref, wo_ref, b_ref, m_ref, r2_ref):
        h = jnp.maximum(p0_ref[...] + p1_ref[...] + r_ref[...], 0.0)
        h = jnp.where(_row_mask((BS, D)), h, 0.0)
        _mm_body(h, wr_ref, wo_ref, b_ref, m_ref, r2_ref)

    return pl.pallas_call(
        body,
        grid=(2,),
        in_specs=[
            pl.BlockSpec((BS, D), lambda i: (i, 0)),
            pl.BlockSpec((BS, D), lambda i: (i, 0)),
            pl.BlockSpec((BS, D), lambda i: (i, 0)),
            pl.BlockSpec((D, D), lambda i: (0, 0)),
            pl.BlockSpec((D, D), lambda i: (0, 0)),
            pl.BlockSpec((1, D), lambda i: (0, 0)),
        ],
        out_specs=[
            pl.BlockSpec((BS, D), lambda i: (i, 0)),
            pl.BlockSpec((BS, D), lambda i: (i, 0)),
        ],
        out_shape=(jax.ShapeDtypeStruct((M_ROWS, D), jnp.float32),
                   jax.ShapeDtypeStruct((N, D), jnp.float32)),
    )(p0, p1, r, W_rel, W_root, b.reshape(1, D))


def _tc_final(p0, p1, r):
    def body(p0_ref, p1_ref, r_ref, o_ref):
        o_ref[...] = p0_ref[...] + p1_ref[...] + r_ref[...]

    return pl.pallas_call(
        body,
        out_shape=jax.ShapeDtypeStruct((N, D), jnp.float32),
    )(p0, p1, r)


def kernel(x, relationsedge_indices_relations, W_rel1, b_rel1, W_root1,
           W_rel2, b_rel2, W_root2):
    ei = relationsedge_indices_relations[-1]
    src, dst = ei[0], ei[1]
    pad = E_PAD - E
    src3d = jnp.concatenate(
        [src, jnp.full((pad,), N, jnp.int32)]).reshape(NW, CPW, CH)
    dst3d = jnp.concatenate(
        [dst, jnp.zeros((pad,), jnp.int32)]).reshape(NCHUNK, 1, CH)

    m1, r1 = _tc_pre(x, W_rel1, W_root1, b_rel1)
    p0, p1 = _seg_sum_sc(m1, src3d, dst3d)
    m2, r2 = _tc_mid(p0, p1, r1, W_rel2, W_root2, b_rel2)
    q0, q1 = _seg_sum_sc(m2, src3d, dst3d)
    out = _tc_final(q0, q1, r2)
    return out.reshape(N, 1, D)
